# baseline (speedup 1.0000x reference)
import functools

import jax
import jax.numpy as jnp
from jax import lax
from jax.experimental import pallas as pl
from jax.experimental.pallas import tpu as pltpu
from jax.experimental.pallas import tpu_sc as plsc

B = 4
N = 10000
E = 640000
BE = B * E
D = 16
EH = 32
EOUT = 22
ACC_W = 24
TSRC_W = 40
BNP = 40960


def _edge_scalars_body(dist_ref, direc_ref, ce_ref, se_ref, ea0_ref, ea1_ref):
    dist = dist_ref[...]
    direc = direc_ref[...]
    dmin = jnp.min(dist)
    dmax = jnp.max(dist)
    cmin = jnp.min(direc)
    cmax = jnp.max(direc)
    inv_d = 1.0 / dist
    ce_ref[...] = jnp.cos(direc) * inv_d
    se_ref[...] = jnp.sin(direc) * inv_d
    ea0_ref[...] = (dist - dmin) / (dmax - dmin)
    ea1_ref[...] = (direc - cmin) / (cmax - cmin)


def _edge_scalars(dist2d, direc2d):
    shp = jax.ShapeDtypeStruct(dist2d.shape, jnp.float32)
    return pl.pallas_call(
        _edge_scalars_body,
        out_shape=(shp, shp, shp, shp),
    )(dist2d, direc2d)


_NT_BLK = 2048


def _node_tables_body(xf_ref, w1s_ref, w1t_ref, b1_ref, wmax_ref, wmin_ref,
                      tsrc_ref, ttgt_ref):
    xf = xf_ref[...]
    p = jnp.dot(xf, w1s_ref[...],
                preferred_element_type=jnp.float32) + b1_ref[...]
    ttgt_ref[...] = jnp.dot(xf, w1t_ref[...],
                            preferred_element_type=jnp.float32)
    wmax = wmax_ref[...]
    wmin = wmin_ref[...]
    speed = xf[:, D - 2] * (wmax[0] - wmin[0]) + wmin[0]
    wd = xf[:, D - 1] * (wmax[1] - wmin[1]) + wmin[1]
    u = speed * jnp.cos(wd)
    v = speed * jnp.sin(wd)
    pad = jnp.zeros((p.shape[0], TSRC_W - EH - 2), jnp.float32)
    tsrc_ref[...] = jnp.concatenate([p, u[:, None], v[:, None], pad], axis=1)


def _node_tables(xf, w1s, w1t, b1, wmax, wmin):
    nblk = BNP // _NT_BLK
    full = lambda *shape: pl.BlockSpec(shape, lambda i: tuple(0 for _ in shape))
    return pl.pallas_call(
        _node_tables_body,
        grid=(nblk,),
        in_specs=[
            pl.BlockSpec((_NT_BLK, D), lambda i: (i, 0)),
            full(D, EH), full(D, EH), full(EH), full(2), full(2),
        ],
        out_specs=(
            pl.BlockSpec((_NT_BLK, TSRC_W), lambda i: (i, 0)),
            pl.BlockSpec((_NT_BLK, EH), lambda i: (i, 0)),
        ),
        out_shape=(
            jax.ShapeDtypeStruct((BNP, TSRC_W), jnp.float32),
            jax.ShapeDtypeStruct((BNP, EH), jnp.float32),
        ),
    )(xf, w1s, w1t, b1, wmax, wmin)


_MB = 2560
_NEB = E // _MB


def _edge_mlp_body(pre_ref, m_ref, w2_ref, b2_ref, out_ref):
    x = pre_ref[...]
    pre1 = jnp.dot(x, m_ref[...], preferred_element_type=jnp.float32)
    h1 = jax.nn.sigmoid(pre1)
    h2 = jax.nn.sigmoid(
        jnp.dot(h1, w2_ref[...], preferred_element_type=jnp.float32)
        + b2_ref[...])
    pad = jnp.zeros((h2.shape[0], ACC_W - EOUT), jnp.float32)
    out_ref[...] = jnp.concatenate([h2, pad], axis=1)


def _edge_mlp(pre, m1, w2, b2):
    full = lambda *shape: pl.BlockSpec(shape, lambda i: tuple(0 for _ in shape))
    return pl.pallas_call(
        _edge_mlp_body,
        grid=(BE // _MB,),
        in_specs=[
            pl.BlockSpec((_MB, TSRC_W), lambda i: (i, 0)),
            full(TSRC_W, EH), full(EH, EOUT), full(EOUT),
        ],
        out_specs=pl.BlockSpec((_MB, ACC_W), lambda i: (i, 0)),
        out_shape=jax.ShapeDtypeStruct((BE, ACC_W), jnp.float32),
    )(pre, m1, w2, b2)


def _finish_body(p_ref, w3_ref, b3_ref, out_ref):
    acc = p_ref[0, :, :EOUT] + p_ref[1, :, :EOUT]
    pre = jnp.dot(acc, w3_ref[...], preferred_element_type=jnp.float32) + b3_ref[...]
    out_ref[...] = jax.nn.sigmoid(pre)


def _finish(partials, w3, b3):
    nblk = BNP // _NT_BLK
    full = lambda *shape: pl.BlockSpec(shape, lambda i: tuple(0 for _ in shape))
    return pl.pallas_call(
        _finish_body,
        grid=(nblk,),
        in_specs=[
            pl.BlockSpec((2, _NT_BLK, ACC_W), lambda i: (0, i, 0)),
            full(EOUT, D), full(D),
        ],
        out_specs=pl.BlockSpec((_NT_BLK, D), lambda i: (i, 0)),
        out_shape=jax.ShapeDtypeStruct((BNP, D), jnp.float32),
    )(partials, w3, b3)


NC = 2
NS = 16
NW = NC * NS
NSL = 1
ES = E // NSL
BES = B * ES
EPW = ES // NW
KP = 80
RPT = BNP // NS

CH1 = 2000
NP1 = CH1 // KP
NOC1 = EPW // CH1
NG = KP // 16

CH2 = 400
NP2 = CH2 // KP
NOC2 = EPW // CH2


def _sc_gather_body(tsrc_hbm, ttgt_hbm, es_hbm, et_hbm, ce_hbm, se_hbm,
                    ea0_hbm, ea1_hbm, pre_hbm,
                    es_v, et_v, ce_v, se_v, ea0_v, ea1_v, gis, git,
                    praw, qraw):
    cid = lax.axis_index("c")
    sid = lax.axis_index("s")
    wid = sid * NC + cid
    iota = lax.iota(jnp.int32, 16)
    c32 = jnp.full((16,), EH, jnp.int32)
    c33 = jnp.full((16,), EH + 1, jnp.int32)
    c34 = jnp.full((16,), EH + 2, jnp.int32)
    c35 = jnp.full((16,), EH + 3, jnp.int32)
    c36 = jnp.full((16,), EH + 4, jnp.int32)

    def _outer(oc, c):
        ebase = wid * EPW + oc * CH1
        pltpu.sync_copy(es_hbm.at[pl.ds(ebase, CH1)], es_v)
        pltpu.sync_copy(et_hbm.at[pl.ds(ebase, CH1)], et_v)
        pltpu.sync_copy(ce_hbm.at[pl.ds(ebase, CH1)], ce_v)
        pltpu.sync_copy(se_hbm.at[pl.ds(ebase, CH1)], se_v)
        pltpu.sync_copy(ea0_hbm.at[pl.ds(ebase, CH1)], ea0_v)
        pltpu.sync_copy(ea1_hbm.at[pl.ds(ebase, CH1)], ea1_v)

        def _piece(p, c2):
            def _batch(b, c3):
                boff = b * N

                def _gidx(g, c4):
                    off = p * KP + g * 16
                    gis[pl.ds(g * 16, 16)] = es_v[pl.ds(off, 16)] + boff
                    git[pl.ds(g * 16, 16)] = et_v[pl.ds(off, 16)] + boff
                    return c4
                lax.fori_loop(0, NG, _gidx, 0)

                pltpu.sync_copy(tsrc_hbm.at[gis], praw)
                pltpu.sync_copy(ttgt_hbm.at[git], qraw)

                def _add(r, c4):
                    praw[r, pl.ds(0, 16)] = (praw[r, pl.ds(0, 16)]
                                             + qraw[r, pl.ds(0, 16)])
                    praw[r, pl.ds(16, 16)] = (praw[r, pl.ds(16, 16)]
                                              + qraw[r, pl.ds(16, 16)])
                    return c4
                lax.fori_loop(0, KP, _add, 0)

                def _ew(g, c4):
                    off = p * KP + g * 16
                    rw = iota + g * 16
                    uu = plsc.load_gather(praw, [rw, c32])
                    vv = plsc.load_gather(praw, [rw, c33])
                    w = uu * ce_v[pl.ds(off, 16)] + vv * se_v[pl.ds(off, 16)]
                    plsc.store_scatter(praw, [rw, c34], ea0_v[pl.ds(off, 16)])
                    plsc.store_scatter(praw, [rw, c35], ea1_v[pl.ds(off, 16)])
                    plsc.store_scatter(praw, [rw, c36], jnp.maximum(w, 0.0))
                    return c4
                lax.fori_loop(0, NG, _ew, 0)

                rowstart = b * E + ebase + p * KP
                pltpu.sync_copy(praw, pre_hbm.at[pl.ds(rowstart, KP)])
                return c3
            lax.fori_loop(0, B, _batch, 0)
            return c2
        lax.fori_loop(0, NP1, _piece, 0)
        return c
    lax.fori_loop(0, NOC1, _outer, 0)


@functools.partial(
    pl.kernel,
    out_type=jax.ShapeDtypeStruct((BE, TSRC_W), jnp.float32),
    mesh=plsc.VectorSubcoreMesh(core_axis_name="c", subcore_axis_name="s",
                                num_cores=NC, num_subcores=NS),
    compiler_params=pltpu.CompilerParams(needs_layout_passes=False,
                                         use_tc_tiling_on_sc=False),
    scratch_types=[
        pltpu.VMEM((CH1,), jnp.int32),
        pltpu.VMEM((CH1,), jnp.int32),
        pltpu.VMEM((CH1,), jnp.float32),
        pltpu.VMEM((CH1,), jnp.float32),
        pltpu.VMEM((CH1,), jnp.float32),
        pltpu.VMEM((CH1,), jnp.float32),
        pltpu.VMEM((KP,), jnp.int32),
        pltpu.VMEM((KP,), jnp.int32),
        pltpu.VMEM((KP, TSRC_W), jnp.float32),
        pltpu.VMEM((KP, EH), jnp.float32),
    ],
)
def _sc_gather(*refs):
    _sc_gather_body(*refs)


def _sc_scatter_body(es_hbm, et_hbm, h2_hbm, out_hbm,
                     es_v, et_v, h2_v, gis, git, updn, zb, acc_sh):
    cid = lax.axis_index("c")
    sid = lax.axis_index("s")
    wid = sid * NC + cid
    zv = jnp.zeros((16,), jnp.float32)

    def _zrow(r, c):
        zb[r, pl.ds(0, 16)] = zv
        zb[r, pl.ds(8, 16)] = zv
        return c
    lax.fori_loop(0, 128, _zrow, 0)

    def _zacc(i, c):
        pltpu.sync_copy(zb, acc_sh.at[pl.ds(sid * RPT + i * 128, 128)])
        return c
    lax.fori_loop(0, RPT // 128, _zacc, 0)
    plsc.subcore_barrier()

    def _outer(oc, c):
        ebase = wid * EPW + oc * CH2
        pltpu.sync_copy(es_hbm.at[pl.ds(ebase, CH2)], es_v)
        pltpu.sync_copy(et_hbm.at[pl.ds(ebase, CH2)], et_v)

        def _batch(b, c3):
            boff = b * N
            pltpu.sync_copy(h2_hbm.at[pl.ds(b * E + ebase, CH2)], h2_v)

            def _piece(p, c2):
                def _gidx(g, c4):
                    off = p * KP + g * 16
                    gis[pl.ds(g * 16, 16)] = es_v[pl.ds(off, 16)] + boff
                    git[pl.ds(g * 16, 16)] = et_v[pl.ds(off, 16)] + boff
                    return c4
                lax.fori_loop(0, NG, _gidx, 0)

                def _neg(r, c4):
                    updn[r, pl.ds(0, 16)] = zv - h2_v[p * KP + r, pl.ds(0, 16)]
                    updn[r, pl.ds(8, 16)] = zv - h2_v[p * KP + r, pl.ds(8, 16)]
                    return c4
                lax.fori_loop(0, KP, _neg, 0)

                pltpu.sync_copy(h2_v.at[pl.ds(p * KP, KP)], acc_sh.at[git],
                                add=True)
                pltpu.sync_copy(updn, acc_sh.at[gis], add=True)
                return c2
            lax.fori_loop(0, NP2, _piece, 0)
            return c3
        lax.fori_loop(0, B, _batch, 0)
        return c
    lax.fori_loop(0, NOC2, _outer, 0)

    plsc.subcore_barrier()
    pltpu.sync_copy(acc_sh.at[pl.ds(sid * RPT, RPT)],
                    out_hbm.at[cid, pl.ds(sid * RPT, RPT)])


@functools.partial(
    pl.kernel,
    out_type=jax.ShapeDtypeStruct((NC, BNP, ACC_W), jnp.float32),
    mesh=plsc.VectorSubcoreMesh(core_axis_name="c", subcore_axis_name="s",
                                num_cores=NC, num_subcores=NS),
    compiler_params=pltpu.CompilerParams(needs_layout_passes=False,
                                         use_tc_tiling_on_sc=False),
    scratch_types=[
        pltpu.VMEM((CH2,), jnp.int32),
        pltpu.VMEM((CH2,), jnp.int32),
        pltpu.VMEM((CH2, ACC_W), jnp.float32),
        pltpu.VMEM((KP,), jnp.int32),
        pltpu.VMEM((KP,), jnp.int32),
        pltpu.VMEM((KP, ACC_W), jnp.float32),
        pltpu.VMEM((128, ACC_W), jnp.float32),
        pltpu.VMEM_SHARED((BNP, ACC_W), jnp.float32),
    ],
)
def _sc_scatter(*refs):
    _sc_scatter_body(*refs)


def kernel(x, edge_index, edge_attr, wind_max, wind_min, W1, b1, W2, b2, W3, b3):
    xf = jnp.pad(x.reshape(B * N, D), ((0, BNP - B * N), (0, 0)))
    dist2d = edge_attr[:, 0].reshape(2500, 256)
    direc2d = edge_attr[:, 1].reshape(2500, 256)
    ce, se, ea0, ea1 = _edge_scalars(dist2d, direc2d)
    ce, se, ea0, ea1 = (a.reshape(E) for a in (ce, se, ea0, ea1))
    tsrc, ttgt = _node_tables(xf, W1[:D, :], W1[D:2 * D, :], b1,
                              wind_max, wind_min)
    es = edge_index[0]
    et = edge_index[1]
    pre = _sc_gather(tsrc, ttgt, es, et, ce, se, ea0, ea1)
    m1 = (jnp.zeros((TSRC_W, EH), jnp.float32)
          .at[:EH].set(jnp.eye(EH, dtype=jnp.float32))
          .at[EH + 2:EH + 5].set(W1[2 * D:, :]))
    h2 = _edge_mlp(pre, m1, W2, b2)
    partials = _sc_scatter(es, et, h2)
    out = _finish(partials, W3, b3)
    return out[:B * N].reshape(B, N, D)


# speedup vs baseline: 1.3276x; 1.3276x over previous
import functools

import jax
import jax.numpy as jnp
from jax import lax
from jax.experimental import pallas as pl
from jax.experimental.pallas import tpu as pltpu
from jax.experimental.pallas import tpu_sc as plsc

B = 4
N = 10000
E = 640000
BE = B * E
D = 16
EH = 32
EOUT = 22
ACC_W = 24
TSRC_W = 40
BNP = 40960


def _edge_scalars_body(dist_ref, direc_ref, ce_ref, se_ref, ea0_ref, ea1_ref):
    dist = dist_ref[...]
    direc = direc_ref[...]
    dmin = jnp.min(dist)
    dmax = jnp.max(dist)
    cmin = jnp.min(direc)
    cmax = jnp.max(direc)
    inv_d = 1.0 / dist
    ce_ref[...] = jnp.cos(direc) * inv_d
    se_ref[...] = jnp.sin(direc) * inv_d
    ea0_ref[...] = (dist - dmin) / (dmax - dmin)
    ea1_ref[...] = (direc - cmin) / (cmax - cmin)


def _edge_scalars(dist2d, direc2d):
    shp = jax.ShapeDtypeStruct(dist2d.shape, jnp.float32)
    return pl.pallas_call(
        _edge_scalars_body,
        out_shape=(shp, shp, shp, shp),
    )(dist2d, direc2d)


_NT_BLK = 2048


def _node_tables_body(xf_ref, w1s_ref, w1t_ref, b1_ref, wmax_ref, wmin_ref,
                      tsrc_ref, ttgt_ref):
    xf = xf_ref[...]
    p = jnp.dot(xf, w1s_ref[...],
                preferred_element_type=jnp.float32) + b1_ref[...]
    ttgt_ref[...] = jnp.dot(xf, w1t_ref[...],
                            preferred_element_type=jnp.float32)
    wmax = wmax_ref[...]
    wmin = wmin_ref[...]
    speed = xf[:, D - 2] * (wmax[0] - wmin[0]) + wmin[0]
    wd = xf[:, D - 1] * (wmax[1] - wmin[1]) + wmin[1]
    u = speed * jnp.cos(wd)
    v = speed * jnp.sin(wd)
    pad = jnp.zeros((p.shape[0], TSRC_W - EH - 2), jnp.float32)
    tsrc_ref[...] = jnp.concatenate([p, u[:, None], v[:, None], pad], axis=1)


def _node_tables(xf, w1s, w1t, b1, wmax, wmin):
    nblk = BNP // _NT_BLK
    full = lambda *shape: pl.BlockSpec(shape, lambda i: tuple(0 for _ in shape))
    return pl.pallas_call(
        _node_tables_body,
        grid=(nblk,),
        in_specs=[
            pl.BlockSpec((_NT_BLK, D), lambda i: (i, 0)),
            full(D, EH), full(D, EH), full(EH), full(2), full(2),
        ],
        out_specs=(
            pl.BlockSpec((_NT_BLK, TSRC_W), lambda i: (i, 0)),
            pl.BlockSpec((_NT_BLK, EH), lambda i: (i, 0)),
        ),
        out_shape=(
            jax.ShapeDtypeStruct((BNP, TSRC_W), jnp.float32),
            jax.ShapeDtypeStruct((BNP, EH), jnp.float32),
        ),
    )(xf, w1s, w1t, b1, wmax, wmin)


_MB = 2560
_NEB = E // _MB


def _edge_mlp_body(pre_ref, m_ref, w2_ref, b2_ref, out_ref):
    x = pre_ref[...]
    pre1 = jnp.dot(x, m_ref[...], preferred_element_type=jnp.float32)
    h1 = jax.nn.sigmoid(pre1)
    h2 = jax.nn.sigmoid(
        jnp.dot(h1, w2_ref[...], preferred_element_type=jnp.float32)
        + b2_ref[...])
    pad = jnp.zeros((h2.shape[0], ACC_W - EOUT), jnp.float32)
    out_ref[...] = jnp.concatenate([h2, pad], axis=1)


def _edge_mlp(pre, m1, w2, b2):
    full = lambda *shape: pl.BlockSpec(shape, lambda i: tuple(0 for _ in shape))
    return pl.pallas_call(
        _edge_mlp_body,
        grid=(BES // _MB,),
        in_specs=[
            pl.BlockSpec((_MB, TSRC_W), lambda i: (i, 0)),
            full(TSRC_W, EH), full(EH, EOUT), full(EOUT),
        ],
        out_specs=pl.BlockSpec((_MB, ACC_W), lambda i: (i, 0)),
        out_shape=jax.ShapeDtypeStruct((BES, ACC_W), jnp.float32),
    )(pre, m1, w2, b2)


def _finish_body(*refs):
    p_refs = refs[:2]
    w3_ref, b3_ref, out_ref = refs[2:]
    acc = p_refs[0][0, :, :EOUT] + p_refs[0][1, :, :EOUT]
    for pr in p_refs[1:]:
        acc = acc + pr[0, :, :EOUT] + pr[1, :, :EOUT]
    pre = jnp.dot(acc, w3_ref[...], preferred_element_type=jnp.float32) + b3_ref[...]
    out_ref[...] = jax.nn.sigmoid(pre)


def _finish(partials_list, w3, b3):
    nblk = BNP // _NT_BLK
    full = lambda *shape: pl.BlockSpec(shape, lambda i: tuple(0 for _ in shape))
    pspec = pl.BlockSpec((2, _NT_BLK, ACC_W), lambda i: (0, i, 0))
    return pl.pallas_call(
        _finish_body,
        grid=(nblk,),
        in_specs=[pspec] * len(partials_list) + [full(EOUT, D), full(D)],
        out_specs=pl.BlockSpec((_NT_BLK, D), lambda i: (i, 0)),
        out_shape=jax.ShapeDtypeStruct((BNP, D), jnp.float32),
    )(*partials_list, w3, b3)


NC = 2
NS = 16
NW = NC * NS
NSL = 2
ES = E // NSL
BES = B * ES
EPW = ES // NW
KP = 80
RPT = BNP // NS

CH1 = 2000
NP1 = CH1 // KP
NOC1 = EPW // CH1
NG = KP // 16

CH2 = 400
NP2 = CH2 // KP
NOC2 = EPW // CH2


def _sc_gather_body(tsrc_hbm, ttgt_hbm, es_hbm, et_hbm, ce_hbm, se_hbm,
                    ea0_hbm, ea1_hbm, pre_hbm,
                    es_v, et_v, ce_v, se_v, ea0_v, ea1_v, gis, git,
                    praw, qraw):
    cid = lax.axis_index("c")
    sid = lax.axis_index("s")
    wid = sid * NC + cid
    iota = lax.iota(jnp.int32, 16)
    c32 = jnp.full((16,), EH, jnp.int32)
    c33 = jnp.full((16,), EH + 1, jnp.int32)
    c34 = jnp.full((16,), EH + 2, jnp.int32)
    c35 = jnp.full((16,), EH + 3, jnp.int32)
    c36 = jnp.full((16,), EH + 4, jnp.int32)

    def _outer(oc, c):
        ebase = wid * EPW + oc * CH1
        pltpu.sync_copy(es_hbm.at[pl.ds(ebase, CH1)], es_v)
        pltpu.sync_copy(et_hbm.at[pl.ds(ebase, CH1)], et_v)
        pltpu.sync_copy(ce_hbm.at[pl.ds(ebase, CH1)], ce_v)
        pltpu.sync_copy(se_hbm.at[pl.ds(ebase, CH1)], se_v)
        pltpu.sync_copy(ea0_hbm.at[pl.ds(ebase, CH1)], ea0_v)
        pltpu.sync_copy(ea1_hbm.at[pl.ds(ebase, CH1)], ea1_v)

        def _piece(p, c2):
            def _batch(b, c3):
                boff = b * N

                def _gidx(g, c4):
                    off = p * KP + g * 16
                    gis[pl.ds(g * 16, 16)] = es_v[pl.ds(off, 16)] + boff
                    git[pl.ds(g * 16, 16)] = et_v[pl.ds(off, 16)] + boff
                    return c4
                lax.fori_loop(0, NG, _gidx, 0)

                pltpu.sync_copy(tsrc_hbm.at[gis], praw)
                pltpu.sync_copy(ttgt_hbm.at[git], qraw)

                def _add(r, c4):
                    praw[r, pl.ds(0, 16)] = (praw[r, pl.ds(0, 16)]
                                             + qraw[r, pl.ds(0, 16)])
                    praw[r, pl.ds(16, 16)] = (praw[r, pl.ds(16, 16)]
                                              + qraw[r, pl.ds(16, 16)])
                    return c4
                lax.fori_loop(0, KP, _add, 0)

                def _ew(g, c4):
                    off = p * KP + g * 16
                    rw = iota + g * 16
                    uu = plsc.load_gather(praw, [rw, c32])
                    vv = plsc.load_gather(praw, [rw, c33])
                    w = uu * ce_v[pl.ds(off, 16)] + vv * se_v[pl.ds(off, 16)]
                    plsc.store_scatter(praw, [rw, c34], ea0_v[pl.ds(off, 16)])
                    plsc.store_scatter(praw, [rw, c35], ea1_v[pl.ds(off, 16)])
                    plsc.store_scatter(praw, [rw, c36], jnp.maximum(w, 0.0))
                    return c4
                lax.fori_loop(0, NG, _ew, 0)

                rowstart = b * ES + ebase + p * KP
                pltpu.sync_copy(praw, pre_hbm.at[pl.ds(rowstart, KP)])
                return c3
            lax.fori_loop(0, B, _batch, 0)
            return c2
        lax.fori_loop(0, NP1, _piece, 0)
        return c
    lax.fori_loop(0, NOC1, _outer, 0)


@functools.partial(
    pl.kernel,
    out_type=jax.ShapeDtypeStruct((BES, TSRC_W), jnp.float32),
    mesh=plsc.VectorSubcoreMesh(core_axis_name="c", subcore_axis_name="s",
                                num_cores=NC, num_subcores=NS),
    compiler_params=pltpu.CompilerParams(needs_layout_passes=False,
                                         use_tc_tiling_on_sc=False),
    scratch_types=[
        pltpu.VMEM((CH1,), jnp.int32),
        pltpu.VMEM((CH1,), jnp.int32),
        pltpu.VMEM((CH1,), jnp.float32),
        pltpu.VMEM((CH1,), jnp.float32),
        pltpu.VMEM((CH1,), jnp.float32),
        pltpu.VMEM((CH1,), jnp.float32),
        pltpu.VMEM((KP,), jnp.int32),
        pltpu.VMEM((KP,), jnp.int32),
        pltpu.VMEM((KP, TSRC_W), jnp.float32),
        pltpu.VMEM((KP, EH), jnp.float32),
    ],
)
def _sc_gather(*refs):
    _sc_gather_body(*refs)


def _sc_scatter_body(es_hbm, et_hbm, h2_hbm, out_hbm,
                     es_v, et_v, h2_v, gis, git, updn, zb, acc_sh):
    cid = lax.axis_index("c")
    sid = lax.axis_index("s")
    wid = sid * NC + cid
    zv = jnp.zeros((16,), jnp.float32)

    def _zrow(r, c):
        zb[r, pl.ds(0, 16)] = zv
        zb[r, pl.ds(8, 16)] = zv
        return c
    lax.fori_loop(0, 128, _zrow, 0)

    def _zacc(i, c):
        pltpu.sync_copy(zb, acc_sh.at[pl.ds(sid * RPT + i * 128, 128)])
        return c
    lax.fori_loop(0, RPT // 128, _zacc, 0)
    plsc.subcore_barrier()

    def _outer(oc, c):
        ebase = wid * EPW + oc * CH2
        pltpu.sync_copy(es_hbm.at[pl.ds(ebase, CH2)], es_v)
        pltpu.sync_copy(et_hbm.at[pl.ds(ebase, CH2)], et_v)

        def _batch(b, c3):
            boff = b * N
            pltpu.sync_copy(h2_hbm.at[pl.ds(b * ES + ebase, CH2)], h2_v)

            def _piece(p, c2):
                def _gidx(g, c4):
                    off = p * KP + g * 16
                    gis[pl.ds(g * 16, 16)] = es_v[pl.ds(off, 16)] + boff
                    git[pl.ds(g * 16, 16)] = et_v[pl.ds(off, 16)] + boff
                    return c4
                lax.fori_loop(0, NG, _gidx, 0)

                def _neg(r, c4):
                    updn[r, pl.ds(0, 16)] = zv - h2_v[p * KP + r, pl.ds(0, 16)]
                    updn[r, pl.ds(8, 16)] = zv - h2_v[p * KP + r, pl.ds(8, 16)]
                    return c4
                lax.fori_loop(0, KP, _neg, 0)

                pltpu.sync_copy(h2_v.at[pl.ds(p * KP, KP)], acc_sh.at[git],
                                add=True)
                pltpu.sync_copy(updn, acc_sh.at[gis], add=True)
                return c2
            lax.fori_loop(0, NP2, _piece, 0)
            return c3
        lax.fori_loop(0, B, _batch, 0)
        return c
    lax.fori_loop(0, NOC2, _outer, 0)

    plsc.subcore_barrier()
    pltpu.sync_copy(acc_sh.at[pl.ds(sid * RPT, RPT)],
                    out_hbm.at[cid, pl.ds(sid * RPT, RPT)])


@functools.partial(
    pl.kernel,
    out_type=jax.ShapeDtypeStruct((NC, BNP, ACC_W), jnp.float32),
    mesh=plsc.VectorSubcoreMesh(core_axis_name="c", subcore_axis_name="s",
                                num_cores=NC, num_subcores=NS),
    compiler_params=pltpu.CompilerParams(needs_layout_passes=False,
                                         use_tc_tiling_on_sc=False),
    scratch_types=[
        pltpu.VMEM((CH2,), jnp.int32),
        pltpu.VMEM((CH2,), jnp.int32),
        pltpu.VMEM((CH2, ACC_W), jnp.float32),
        pltpu.VMEM((KP,), jnp.int32),
        pltpu.VMEM((KP,), jnp.int32),
        pltpu.VMEM((KP, ACC_W), jnp.float32),
        pltpu.VMEM((128, ACC_W), jnp.float32),
        pltpu.VMEM_SHARED((BNP, ACC_W), jnp.float32),
    ],
)
def _sc_scatter(*refs):
    _sc_scatter_body(*refs)


def kernel(x, edge_index, edge_attr, wind_max, wind_min, W1, b1, W2, b2, W3, b3):
    xf = jnp.pad(x.reshape(B * N, D), ((0, BNP - B * N), (0, 0)))
    dist2d = edge_attr[:, 0].reshape(2500, 256)
    direc2d = edge_attr[:, 1].reshape(2500, 256)
    ce, se, ea0, ea1 = _edge_scalars(dist2d, direc2d)
    ce, se, ea0, ea1 = (a.reshape(E) for a in (ce, se, ea0, ea1))
    tsrc, ttgt = _node_tables(xf, W1[:D, :], W1[D:2 * D, :], b1,
                              wind_max, wind_min)
    es = edge_index[0]
    et = edge_index[1]
    m1 = (jnp.zeros((TSRC_W, EH), jnp.float32)
          .at[:EH].set(jnp.eye(EH, dtype=jnp.float32))
          .at[EH + 2:EH + 5].set(W1[2 * D:, :]))
    sl = lambda a, s: a[s * ES:(s + 1) * ES]
    pres = [_sc_gather(tsrc, ttgt, sl(es, s), sl(et, s), sl(ce, s),
                       sl(se, s), sl(ea0, s), sl(ea1, s))
            for s in range(NSL)]
    h2s = [_edge_mlp(p, m1, W2, b2) for p in pres]
    partials = [_sc_scatter(sl(es, s), sl(et, s), h2s[s]) for s in range(NSL)]
    out = _finish(partials, W3, b3)
    return out[:B * N].reshape(B, N, D)


# speedup vs baseline: 1.6125x; 1.2146x over previous
import functools

import jax
import jax.numpy as jnp
from jax import lax
from jax.experimental import pallas as pl
from jax.experimental.pallas import tpu as pltpu
from jax.experimental.pallas import tpu_sc as plsc

B = 4
N = 10000
E = 640000
BE = B * E
D = 16
EH = 32
EOUT = 22
ACC_W = 24
TSRC_W = 40
BNP = 40960


def _edge_scalars_body(dist_ref, direc_ref, ce_ref, se_ref, ea0_ref, ea1_ref):
    dist = dist_ref[...]
    direc = direc_ref[...]
    dmin = jnp.min(dist)
    dmax = jnp.max(dist)
    cmin = jnp.min(direc)
    cmax = jnp.max(direc)
    inv_d = 1.0 / dist
    ce_ref[...] = jnp.cos(direc) * inv_d
    se_ref[...] = jnp.sin(direc) * inv_d
    ea0_ref[...] = (dist - dmin) / (dmax - dmin)
    ea1_ref[...] = (direc - cmin) / (cmax - cmin)


def _edge_scalars(dist2d, direc2d):
    shp = jax.ShapeDtypeStruct(dist2d.shape, jnp.float32)
    return pl.pallas_call(
        _edge_scalars_body,
        out_shape=(shp, shp, shp, shp),
    )(dist2d, direc2d)


_NT_BLK = 2048


def _node_tables_body(xf_ref, w1s_ref, w1t_ref, b1_ref, wmax_ref, wmin_ref,
                      tsrc_ref, ttgt_ref):
    xf = xf_ref[...]
    p = jnp.dot(xf, w1s_ref[...],
                preferred_element_type=jnp.float32) + b1_ref[...]
    ttgt_ref[...] = jnp.dot(xf, w1t_ref[...],
                            preferred_element_type=jnp.float32)
    wmax = wmax_ref[...]
    wmin = wmin_ref[...]
    speed = xf[:, D - 2] * (wmax[0] - wmin[0]) + wmin[0]
    wd = xf[:, D - 1] * (wmax[1] - wmin[1]) + wmin[1]
    u = speed * jnp.cos(wd)
    v = speed * jnp.sin(wd)
    pad = jnp.zeros((p.shape[0], TSRC_W - EH - 2), jnp.float32)
    tsrc_ref[...] = jnp.concatenate([p, u[:, None], v[:, None], pad], axis=1)


def _node_tables(xf, w1s, w1t, b1, wmax, wmin):
    nblk = BNP // _NT_BLK
    full = lambda *shape: pl.BlockSpec(shape, lambda i: tuple(0 for _ in shape))
    return pl.pallas_call(
        _node_tables_body,
        grid=(nblk,),
        in_specs=[
            pl.BlockSpec((_NT_BLK, D), lambda i: (i, 0)),
            full(D, EH), full(D, EH), full(EH), full(2), full(2),
        ],
        out_specs=(
            pl.BlockSpec((_NT_BLK, TSRC_W), lambda i: (i, 0)),
            pl.BlockSpec((_NT_BLK, EH), lambda i: (i, 0)),
        ),
        out_shape=(
            jax.ShapeDtypeStruct((BNP, TSRC_W), jnp.float32),
            jax.ShapeDtypeStruct((BNP, EH), jnp.float32),
        ),
    )(xf, w1s, w1t, b1, wmax, wmin)


_MB = 2560
_NEB = E // _MB


def _edge_mlp_body(pre_ref, m_ref, w2_ref, b2_ref, out_ref):
    x = pre_ref[...]
    pre1 = jnp.dot(x, m_ref[...], preferred_element_type=jnp.float32)
    h1 = jax.nn.sigmoid(pre1)
    h2 = jax.nn.sigmoid(
        jnp.dot(h1, w2_ref[...], preferred_element_type=jnp.float32)
        + b2_ref[...])
    pad = jnp.zeros((h2.shape[0], ACC_W - EOUT), jnp.float32)
    out_ref[...] = jnp.concatenate([h2, pad], axis=1)


def _edge_mlp(pre, m1, w2, b2):
    full = lambda *shape: pl.BlockSpec(shape, lambda i: tuple(0 for _ in shape))
    return pl.pallas_call(
        _edge_mlp_body,
        grid=(BES // _MB,),
        in_specs=[
            pl.BlockSpec((_MB, TSRC_W), lambda i: (i, 0)),
            full(TSRC_W, EH), full(EH, EOUT), full(EOUT),
        ],
        out_specs=pl.BlockSpec((_MB, ACC_W), lambda i: (i, 0)),
        out_shape=jax.ShapeDtypeStruct((BES, ACC_W), jnp.float32),
    )(pre, m1, w2, b2)


def _finish_body(*refs):
    p_refs = refs[:-3]
    w3_ref, b3_ref, out_ref = refs[-3:]
    acc = p_refs[0][0, :, :EOUT] + p_refs[0][1, :, :EOUT]
    for pr in p_refs[1:]:
        acc = acc + pr[0, :, :EOUT] + pr[1, :, :EOUT]
    pre = jnp.dot(acc, w3_ref[...], preferred_element_type=jnp.float32) + b3_ref[...]
    out_ref[...] = jax.nn.sigmoid(pre)


def _finish(partials_list, w3, b3):
    nblk = BNP // _NT_BLK
    full = lambda *shape: pl.BlockSpec(shape, lambda i: tuple(0 for _ in shape))
    pspec = pl.BlockSpec((2, _NT_BLK, ACC_W), lambda i: (0, i, 0))
    return pl.pallas_call(
        _finish_body,
        grid=(nblk,),
        in_specs=[pspec] * len(partials_list) + [full(EOUT, D), full(D)],
        out_specs=pl.BlockSpec((_NT_BLK, D), lambda i: (i, 0)),
        out_shape=jax.ShapeDtypeStruct((BNP, D), jnp.float32),
    )(*partials_list, w3, b3)


NC = 2
NS = 16
NW = NC * NS
NSL = 5
ES = E // NSL
BES = B * ES
EPW = ES // NW
KP = 80
RPT = BNP // NS

CH1 = 2000
NP1 = CH1 // KP
NOC1 = EPW // CH1
NG = KP // 16

CH2 = 400
NP2 = CH2 // KP
NOC2 = EPW // CH2


def _sc_gather_body(tsrc_hbm, ttgt_hbm, es_hbm, et_hbm, ce_hbm, se_hbm,
                    ea0_hbm, ea1_hbm, pre_hbm,
                    es_v, et_v, ce_v, se_v, ea0_v, ea1_v, gis, git,
                    praw, qraw):
    cid = lax.axis_index("c")
    sid = lax.axis_index("s")
    wid = sid * NC + cid
    iota = lax.iota(jnp.int32, 16)
    c32 = jnp.full((16,), EH, jnp.int32)
    c33 = jnp.full((16,), EH + 1, jnp.int32)
    c34 = jnp.full((16,), EH + 2, jnp.int32)
    c35 = jnp.full((16,), EH + 3, jnp.int32)
    c36 = jnp.full((16,), EH + 4, jnp.int32)

    def _outer(oc, c):
        ebase = wid * EPW + oc * CH1
        pltpu.sync_copy(es_hbm.at[pl.ds(ebase, CH1)], es_v)
        pltpu.sync_copy(et_hbm.at[pl.ds(ebase, CH1)], et_v)
        pltpu.sync_copy(ce_hbm.at[pl.ds(ebase, CH1)], ce_v)
        pltpu.sync_copy(se_hbm.at[pl.ds(ebase, CH1)], se_v)
        pltpu.sync_copy(ea0_hbm.at[pl.ds(ebase, CH1)], ea0_v)
        pltpu.sync_copy(ea1_hbm.at[pl.ds(ebase, CH1)], ea1_v)

        def _piece(p, c2):
            def _batch(b, c3):
                boff = b * N

                def _gidx(g, c4):
                    off = p * KP + g * 16
                    gis[pl.ds(g * 16, 16)] = es_v[pl.ds(off, 16)] + boff
                    git[pl.ds(g * 16, 16)] = et_v[pl.ds(off, 16)] + boff
                    return c4
                lax.fori_loop(0, NG, _gidx, 0)

                pltpu.sync_copy(tsrc_hbm.at[gis], praw)
                pltpu.sync_copy(ttgt_hbm.at[git], qraw)

                def _add(r, c4):
                    praw[r, pl.ds(0, 16)] = (praw[r, pl.ds(0, 16)]
                                             + qraw[r, pl.ds(0, 16)])
                    praw[r, pl.ds(16, 16)] = (praw[r, pl.ds(16, 16)]
                                              + qraw[r, pl.ds(16, 16)])
                    return c4
                lax.fori_loop(0, KP, _add, 0)

                def _ew(g, c4):
                    off = p * KP + g * 16
                    rw = iota + g * 16
                    uu = plsc.load_gather(praw, [rw, c32])
                    vv = plsc.load_gather(praw, [rw, c33])
                    w = uu * ce_v[pl.ds(off, 16)] + vv * se_v[pl.ds(off, 16)]
                    plsc.store_scatter(praw, [rw, c34], ea0_v[pl.ds(off, 16)])
                    plsc.store_scatter(praw, [rw, c35], ea1_v[pl.ds(off, 16)])
                    plsc.store_scatter(praw, [rw, c36], jnp.maximum(w, 0.0))
                    return c4
                lax.fori_loop(0, NG, _ew, 0)

                rowstart = b * ES + ebase + p * KP
                pltpu.sync_copy(praw, pre_hbm.at[pl.ds(rowstart, KP)])
                return c3
            lax.fori_loop(0, B, _batch, 0)
            return c2
        lax.fori_loop(0, NP1, _piece, 0)
        return c
    lax.fori_loop(0, NOC1, _outer, 0)


@functools.partial(
    pl.kernel,
    out_type=jax.ShapeDtypeStruct((BES, TSRC_W), jnp.float32),
    mesh=plsc.VectorSubcoreMesh(core_axis_name="c", subcore_axis_name="s",
                                num_cores=NC, num_subcores=NS),
    compiler_params=pltpu.CompilerParams(needs_layout_passes=False,
                                         use_tc_tiling_on_sc=False),
    scratch_types=[
        pltpu.VMEM((CH1,), jnp.int32),
        pltpu.VMEM((CH1,), jnp.int32),
        pltpu.VMEM((CH1,), jnp.float32),
        pltpu.VMEM((CH1,), jnp.float32),
        pltpu.VMEM((CH1,), jnp.float32),
        pltpu.VMEM((CH1,), jnp.float32),
        pltpu.VMEM((KP,), jnp.int32),
        pltpu.VMEM((KP,), jnp.int32),
        pltpu.VMEM((KP, TSRC_W), jnp.float32),
        pltpu.VMEM((KP, EH), jnp.float32),
    ],
)
def _sc_gather(*refs):
    _sc_gather_body(*refs)


def _sc_scatter_body(es_hbm, et_hbm, h2_hbm, out_hbm,
                     es_v, et_v, h2_v, gis, git, updn, zb, acc_sh):
    cid = lax.axis_index("c")
    sid = lax.axis_index("s")
    wid = sid * NC + cid
    zv = jnp.zeros((16,), jnp.float32)

    def _zrow(r, c):
        zb[r, pl.ds(0, 16)] = zv
        zb[r, pl.ds(8, 16)] = zv
        return c
    lax.fori_loop(0, 128, _zrow, 0)

    def _zacc(i, c):
        pltpu.sync_copy(zb, acc_sh.at[pl.ds(sid * RPT + i * 128, 128)])
        return c
    lax.fori_loop(0, RPT // 128, _zacc, 0)
    plsc.subcore_barrier()

    def _outer(oc, c):
        ebase = wid * EPW + oc * CH2
        pltpu.sync_copy(es_hbm.at[pl.ds(ebase, CH2)], es_v)
        pltpu.sync_copy(et_hbm.at[pl.ds(ebase, CH2)], et_v)

        def _batch(b, c3):
            boff = b * N
            pltpu.sync_copy(h2_hbm.at[pl.ds(b * ES + ebase, CH2)], h2_v)

            def _piece(p, c2):
                def _gidx(g, c4):
                    off = p * KP + g * 16
                    gis[pl.ds(g * 16, 16)] = es_v[pl.ds(off, 16)] + boff
                    git[pl.ds(g * 16, 16)] = et_v[pl.ds(off, 16)] + boff
                    return c4
                lax.fori_loop(0, NG, _gidx, 0)

                def _neg(r, c4):
                    updn[r, pl.ds(0, 16)] = zv - h2_v[p * KP + r, pl.ds(0, 16)]
                    updn[r, pl.ds(8, 16)] = zv - h2_v[p * KP + r, pl.ds(8, 16)]
                    return c4
                lax.fori_loop(0, KP, _neg, 0)

                pltpu.sync_copy(h2_v.at[pl.ds(p * KP, KP)], acc_sh.at[git],
                                add=True)
                pltpu.sync_copy(updn, acc_sh.at[gis], add=True)
                return c2
            lax.fori_loop(0, NP2, _piece, 0)
            return c3
        lax.fori_loop(0, B, _batch, 0)
        return c
    lax.fori_loop(0, NOC2, _outer, 0)

    plsc.subcore_barrier()
    pltpu.sync_copy(acc_sh.at[pl.ds(sid * RPT, RPT)],
                    out_hbm.at[cid, pl.ds(sid * RPT, RPT)])


@functools.partial(
    pl.kernel,
    out_type=jax.ShapeDtypeStruct((NC, BNP, ACC_W), jnp.float32),
    mesh=plsc.VectorSubcoreMesh(core_axis_name="c", subcore_axis_name="s",
                                num_cores=NC, num_subcores=NS),
    compiler_params=pltpu.CompilerParams(needs_layout_passes=False,
                                         use_tc_tiling_on_sc=False),
    scratch_types=[
        pltpu.VMEM((CH2,), jnp.int32),
        pltpu.VMEM((CH2,), jnp.int32),
        pltpu.VMEM((CH2, ACC_W), jnp.float32),
        pltpu.VMEM((KP,), jnp.int32),
        pltpu.VMEM((KP,), jnp.int32),
        pltpu.VMEM((KP, ACC_W), jnp.float32),
        pltpu.VMEM((128, ACC_W), jnp.float32),
        pltpu.VMEM_SHARED((BNP, ACC_W), jnp.float32),
    ],
)
def _sc_scatter(*refs):
    _sc_scatter_body(*refs)


def kernel(x, edge_index, edge_attr, wind_max, wind_min, W1, b1, W2, b2, W3, b3):
    xf = jnp.pad(x.reshape(B * N, D), ((0, BNP - B * N), (0, 0)))
    dist2d = edge_attr[:, 0].reshape(2500, 256)
    direc2d = edge_attr[:, 1].reshape(2500, 256)
    ce, se, ea0, ea1 = _edge_scalars(dist2d, direc2d)
    ce, se, ea0, ea1 = (a.reshape(E) for a in (ce, se, ea0, ea1))
    tsrc, ttgt = _node_tables(xf, W1[:D, :], W1[D:2 * D, :], b1,
                              wind_max, wind_min)
    es = edge_index[0]
    et = edge_index[1]
    m1 = (jnp.zeros((TSRC_W, EH), jnp.float32)
          .at[:EH].set(jnp.eye(EH, dtype=jnp.float32))
          .at[EH + 2:EH + 5].set(W1[2 * D:, :]))
    sl = lambda a, s: a[s * ES:(s + 1) * ES]
    pres = [_sc_gather(tsrc, ttgt, sl(es, s), sl(et, s), sl(ce, s),
                       sl(se, s), sl(ea0, s), sl(ea1, s))
            for s in range(NSL)]
    h2s = [_edge_mlp(p, m1, W2, b2) for p in pres]
    partials = [_sc_scatter(sl(es, s), sl(et, s), h2s[s]) for s in range(NSL)]
    out = _finish(partials, W3, b3)
    return out[:B * N].reshape(B, N, D)
